# baseline (speedup 1.0000x reference)
import functools

import jax
import jax.numpy as jnp
from jax import lax
from jax.experimental import pallas as pl
from jax.experimental.pallas import tpu as pltpu
from jax.experimental.pallas import tpu_sc as plsc

D = 32
NW = 32
C = 512
K = 1
CH = K * C


@functools.lru_cache(maxsize=None)
def _make_kernel(total: int):
    per_w = total // NW
    nch = per_w // C
    nit = nch // (2 * K)
    mesh = plsc.VectorSubcoreMesh(core_axis_name="c", subcore_axis_name="s")

    @functools.partial(
        pl.kernel,
        mesh=mesh,
        compiler_params=pltpu.CompilerParams(
            use_tc_tiling_on_sc=False, skip_device_barrier=True),
        out_type=(
            jax.ShapeDtypeStruct((total, D), jnp.float32),
            jax.ShapeDtypeStruct((total, D), jnp.float32),
        ),
        scratch_types=[
            pltpu.VMEM((nch, C), jnp.int32),
            pltpu.VMEM((CH, D), jnp.float32),
            pltpu.VMEM((CH, D), jnp.float32),
            pltpu.VMEM((CH, D), jnp.float32),
            pltpu.VMEM((CH, D), jnp.float32),
            pltpu.SemaphoreType.DMA,
            pltpu.SemaphoreType.DMA,
            pltpu.SemaphoreType.DMA,
            pltpu.SemaphoreType.DMA,
        ],
    )
    def k(ids_hbm, wre_hbm, wim_hbm, ore_hbm, oim_hbm,
          idx_v, bre0, bim0, bre1, bim1, sem_g0, sem_g1, sem_w0, sem_w1):
        wid = lax.axis_index("s") * 2 + lax.axis_index("c")
        pltpu.sync_copy(ids_hbm.at[wid], idx_v)
        base = wid * per_w

        def drain_writes(bre, bim, sem):
            pltpu.make_async_copy(bre, ore_hbm.at[pl.ds(0, CH)], sem).wait()
            pltpu.make_async_copy(bim, oim_hbm.at[pl.ds(0, CH)], sem).wait()

        def fire_gathers(c0, bre, bim, sem):
            cps = []
            for i in range(K):
                idx = idx_v.at[c0 + i]
                cps.append(pltpu.async_copy(
                    wre_hbm.at[idx], bre.at[pl.ds(i * C, C)], sem))
                cps.append(pltpu.async_copy(
                    wim_hbm.at[idx], bim.at[pl.ds(i * C, C)], sem))
            return cps

        def body(jj, carry):
            c0 = 2 * K * jj
            c1 = c0 + K

            @pl.when(jj > 0)
            def _():
                drain_writes(bre0, bim0, sem_w0)
            g0 = fire_gathers(c0, bre0, bim0, sem_g0)

            @pl.when(jj > 0)
            def _():
                drain_writes(bre1, bim1, sem_w1)
            g1 = fire_gathers(c1, bre1, bim1, sem_g1)

            for cp in g0:
                cp.wait()
            pltpu.async_copy(bre0, ore_hbm.at[pl.ds(base + c0 * C, CH)], sem_w0)
            pltpu.async_copy(bim0, oim_hbm.at[pl.ds(base + c0 * C, CH)], sem_w0)

            for cp in g1:
                cp.wait()
            pltpu.async_copy(bre1, ore_hbm.at[pl.ds(base + c1 * C, CH)], sem_w1)
            pltpu.async_copy(bim1, oim_hbm.at[pl.ds(base + c1 * C, CH)], sem_w1)
            return carry

        lax.fori_loop(0, nit, body, 0)
        drain_writes(bre0, bim0, sem_w0)
        drain_writes(bre1, bim1, sem_w1)

    return k


def kernel(input_ids, W_re, W_im):
    b, s = input_ids.shape
    total = b * s
    ids3 = input_ids.reshape(NW, total // NW // C, C)
    out_re, out_im = _make_kernel(total)(ids3, W_re, W_im)
    return (out_re.reshape(b, s, D), out_im.reshape(b, s, D))


# speedup vs baseline: 1.0328x; 1.0328x over previous
import functools

import jax
import jax.numpy as jnp
from jax import lax
from jax.experimental import pallas as pl
from jax.experimental.pallas import tpu as pltpu
from jax.experimental.pallas import tpu_sc as plsc

D = 32
NW = 32
C = 512


@functools.lru_cache(maxsize=None)
def _make_kernel(total: int):
    per_w = total // NW
    nch = per_w // C
    nit = nch // 2
    mesh = plsc.VectorSubcoreMesh(core_axis_name="c", subcore_axis_name="s")

    @functools.partial(
        pl.kernel,
        mesh=mesh,
        compiler_params=pltpu.CompilerParams(use_tc_tiling_on_sc=False),
        out_type=jax.ShapeDtypeStruct((total, D), jnp.float32),
        scratch_types=[
            pltpu.VMEM((nch, C), jnp.int32),
            pltpu.VMEM((C, D), jnp.float32),
            pltpu.VMEM((C, D), jnp.float32),
            pltpu.SemaphoreType.DMA,
            pltpu.SemaphoreType.DMA,
            pltpu.SemaphoreType.DMA,
            pltpu.SemaphoreType.DMA,
        ],
    )
    def k(ids_hbm, w_hbm, o_hbm,
          idx_v, b0, b1, sem_g0, sem_g1, sem_w0, sem_w1):
        wid = lax.axis_index("s") * 2 + lax.axis_index("c")
        pltpu.sync_copy(ids_hbm.at[wid], idx_v)
        base = wid * per_w

        def drain_write(buf, sem):
            pltpu.make_async_copy(buf, o_hbm.at[pl.ds(0, C)], sem).wait()

        def body(jj, carry):
            c0 = 2 * jj
            c1 = c0 + 1

            @pl.when(jj > 0)
            def _():
                drain_write(b0, sem_w0)
            g0 = pltpu.async_copy(w_hbm.at[idx_v.at[c0]], b0, sem_g0)

            @pl.when(jj > 0)
            def _():
                drain_write(b1, sem_w1)
            g1 = pltpu.async_copy(w_hbm.at[idx_v.at[c1]], b1, sem_g1)

            g0.wait()
            pltpu.async_copy(b0, o_hbm.at[pl.ds(base + c0 * C, C)], sem_w0)
            g1.wait()
            pltpu.async_copy(b1, o_hbm.at[pl.ds(base + c1 * C, C)], sem_w1)
            return carry

        lax.fori_loop(0, nit, body, 0)
        drain_write(b0, sem_w0)
        drain_write(b1, sem_w1)

    return k


def kernel(input_ids, W_re, W_im):
    b, s = input_ids.shape
    total = b * s
    ids3 = input_ids.reshape(NW, total // NW // C, C)
    fn = _make_kernel(total)
    out_re = fn(ids3, W_re)
    out_im = fn(ids3, W_im)
    return (out_re.reshape(b, s, D), out_im.reshape(b, s, D))
